# baseline (speedup 1.0000x reference)
import functools

import numpy as np

import jax
import jax.numpy as jnp
from jax import lax
from jax.experimental import pallas as pl
from jax.experimental.pallas import tpu as pltpu
from jax.experimental.pallas import tpu_sc as plsc

_LANES = 16
_NC = 2
_NS = 16
_NW = _NC * _NS
_CHUNK = 80
_ZBLK = 16


def _spmm_body(n_pad, d, n_chunks, rows_per_tile,
               x_hbm, sd_hbm, w_hbm, out_hbm,
               sd_v0, sd_v1, dst_v0, dst_v1, w_v0, w_v1,
               rows_v0, rows_v1, sc_v0, sc_v1, zrow_v, acc_sh,
               sem_sw0, sem_sw1,
               sem_row0, sem_row1, sem_sc0, sem_sc1):
  sd_v = (sd_v0, sd_v1)
  dst_v = (dst_v0, dst_v1)
  w_v = (w_v0, w_v1)
  rows_v = (rows_v0, rows_v1)
  sc_v = (sc_v0, sc_v1)
  sem_sw = (sem_sw0, sem_sw1)
  sem_row = (sem_row0, sem_row1)
  sem_sc = (sem_sc0, sem_sc1)

  cid = lax.axis_index("c")
  sid = lax.axis_index("s")
  wid = sid * _NC + cid
  base = wid * n_chunks

  def sw_copies(c, p):
    off2 = pl.ds((base + c) * 2 * _CHUNK, 2 * _CHUNK)
    off = pl.ds((base + c) * _CHUNK, _CHUNK)
    return (
        pltpu.make_async_copy(sd_hbm.at[off2], sd_v[p], sem_sw[p]),
        pltpu.make_async_copy(w_hbm.at[off], w_v[p], sem_sw[p]),
    )

  def copy_dst(p):
    for k in range(_CHUNK // _LANES):
      sl = pl.ds(k * _LANES, _LANES)
      dst_v[p][sl] = sd_v[p][pl.ds(_CHUNK + k * _LANES, _LANES)]

  def gather_copy(p):
    idx = sd_v[p].at[pl.ds(0, _CHUNK)]
    return pltpu.make_async_copy(x_hbm.at[idx], rows_v[p], sem_row[p])

  def scatter_copy(p):
    return pltpu.make_async_copy(sc_v[p], acc_sh.at[dst_v[p]], sem_sc[p])

  zeros = jnp.zeros((_LANES,), jnp.float32)

  def zfill(r, carry):
    for j in range(d // _LANES):
      zrow_v[r, j * _LANES:(j + 1) * _LANES] = zeros
    return carry

  lax.fori_loop(0, _ZBLK, zfill, 0)

  def zero_copy(k):
    sl = pl.ds(sid * rows_per_tile + k * _ZBLK, _ZBLK)
    return pltpu.make_async_copy(zrow_v, acc_sh.at[sl], sem_sw0)

  def zstart(k, carry):
    zero_copy(k).start()
    return carry

  def zdrain(k, carry):
    zero_copy(k).wait()
    return carry

  lax.fori_loop(0, rows_per_tile // _ZBLK, zstart, 0)
  lax.fori_loop(0, rows_per_tile // _ZBLK, zdrain, 0)
  plsc.subcore_barrier()

  def scale(p):
    def group_body(eg, c2):
      wvec = w_v[p][pl.ds(eg * _LANES, _LANES)]
      for l in range(_LANES):
        e = eg * _LANES + l
        w = wvec[l]
        for j in range(d // _LANES):
          sl = pl.ds(j * _LANES, _LANES)
          sc_v[p][e, sl] = rows_v[p][e, sl] * w
      return c2

    lax.fori_loop(0, _CHUNK // _LANES, group_body, 0)

  def _maybe(static, cond, fn):
    if static:
      if cond:
        fn()
    else:
      pl.when(cond)(fn)

  def iteration(g, p, first, has_next, has_next2):
    static = isinstance(g, int)
    q = 1 - p
    if not first:
      _maybe(static, g >= 2, lambda: scatter_copy(p).wait())
    gather_copy(p).wait()
    copy_dst(p)

    def do_next():
      for cp in sw_copies(g + 1, q):
        cp.wait()
      gather_copy(q).start()

    if has_next:
      _maybe(static, g + 1 < n_chunks, do_next)
    scale(p)
    scatter_copy(p).start(add=True)

    def do_next2():
      for cp in sw_copies(g + 2, p):
        cp.start()

    if has_next2:
      _maybe(static, g + 2 < n_chunks, do_next2)

  for cp in sw_copies(0, 0):
    cp.start()
  for cp in sw_copies(1, 1):
    cp.start()
  for cp in sw_copies(0, 0):
    cp.wait()
  gather_copy(0).start()

  def pair_body(t, carry):
    g = 2 * t
    iteration(g, 0, False, True, True)
    iteration(g + 1, 1, False, True, True)
    return carry

  lax.fori_loop(0, n_chunks // 2, pair_body, 0)
  if n_chunks % 2:
    iteration(n_chunks - 1, 0, False, False, False)
  scatter_copy(1 - (n_chunks - 1) % 2).wait()
  scatter_copy((n_chunks - 1) % 2).wait()
  plsc.subcore_barrier()

  osl = pl.ds(sid * rows_per_tile, rows_per_tile)
  pltpu.sync_copy(acc_sh.at[osl], out_hbm.at[cid].at[osl])


def _pad_to(n, q):
  return ((n + q - 1) // q) * q


def _sc_aggregate(xb, sd, ew, n_nodes, d):
  e_pad = ew.shape[0]
  n_chunks = e_pad // (_NW * _CHUNK)
  n_pad = _pad_to(n_nodes, _ZBLK * _NS)
  rows_per_tile = n_pad // _NS

  mesh = plsc.VectorSubcoreMesh(core_axis_name="c", subcore_axis_name="s")
  kern = pl.kernel(
      functools.partial(_spmm_body, n_pad, d, n_chunks, rows_per_tile),
      mesh=mesh,
      out_type=jax.ShapeDtypeStruct((_NC, n_pad, d), jnp.float32),
      scratch_types=[
          pltpu.VMEM((2 * _CHUNK,), jnp.int32),
          pltpu.VMEM((2 * _CHUNK,), jnp.int32),
          pltpu.VMEM((_CHUNK,), jnp.int32),
          pltpu.VMEM((_CHUNK,), jnp.int32),
          pltpu.VMEM((_CHUNK,), jnp.float32),
          pltpu.VMEM((_CHUNK,), jnp.float32),
          pltpu.VMEM((_CHUNK, d), jnp.float32),
          pltpu.VMEM((_CHUNK, d), jnp.float32),
          pltpu.VMEM((_CHUNK, d), jnp.float32),
          pltpu.VMEM((_CHUNK, d), jnp.float32),
          pltpu.VMEM((_ZBLK, d), jnp.float32),
          pltpu.VMEM_SHARED((n_pad, d), jnp.float32),
          pltpu.SemaphoreType.DMA,
          pltpu.SemaphoreType.DMA,
          pltpu.SemaphoreType.DMA,
          pltpu.SemaphoreType.DMA,
          pltpu.SemaphoreType.DMA,
          pltpu.SemaphoreType.DMA,
      ],
  )
  return kern(xb, sd, ew)


def _combine_matmul_kernel(p_ref, w_ref, b_ref, o_ref):
  s = p_ref[0] + p_ref[1]
  o_ref[...] = (
      jnp.dot(s, w_ref[...], preferred_element_type=jnp.float32) + b_ref[...]
  )


def _combine_matmul(partials, W, b, n_nodes):
  d_in = partials.shape[2]
  d_out = W.shape[1]
  blk = 1000
  grid = n_nodes // blk
  return pl.pallas_call(
      _combine_matmul_kernel,
      grid=(grid,),
      in_specs=[
          pl.BlockSpec((2, blk, d_in), lambda i: (0, i, 0)),
          pl.BlockSpec((d_in, d_out), lambda i: (0, 0)),
          pl.BlockSpec((1, d_out), lambda i: (0, 0)),
      ],
      out_specs=pl.BlockSpec((blk, d_out), lambda i: (i, 0)),
      out_shape=jax.ShapeDtypeStruct((n_nodes, d_out), jnp.float32),
  )(partials, W, b.reshape(1, d_out))


def kernel(x, edge_index, edge_weight, W, b):
  n_edges = edge_weight.shape[0]
  quantum = _NW * _CHUNK
  pad = _pad_to(n_edges, quantum) - n_edges
  if pad:
    edge_index = jnp.pad(edge_index, ((0, 0), (0, pad)))
    edge_weight = jnp.pad(edge_weight, (0, pad))

  dst = edge_index[0].astype(jnp.int32)
  src = edge_index[1].astype(jnp.int32)
  ew = edge_weight.astype(jnp.float32)
  nct = src.shape[0] // _CHUNK
  sd = jnp.stack([src.reshape(nct, _CHUNK), dst.reshape(nct, _CHUNK)],
                 axis=1).reshape(-1)

  partials = _sc_aggregate(x, sd, ew, x.shape[0], x.shape[1])
  return _combine_matmul(partials, W, b, x.shape[0])


# speedup vs baseline: 1.1171x; 1.1171x over previous
import functools

import jax
import jax.numpy as jnp
from jax import lax
from jax.experimental import pallas as pl
from jax.experimental.pallas import tpu as pltpu
from jax.experimental.pallas import tpu_sc as plsc

_LANES = 16
_NC = 2
_NS = 16
_NW = _NC * _NS
_CHUNK = 80
_ZBLK = 16


def _spmm_body(n_pad, d, n_chunks, rows_per_tile,
               x_hbm, src_hbm, dst_hbm, w_hbm, out_hbm,
               src_v0, src_v1, dst_v0, dst_v1, w_v0, w_v1,
               rows_v0, rows_v1, sc_v0, sc_v1, zrow_v, acc_sh,
               sem_sw0, sem_sw1, sem_dst0, sem_dst1,
               sem_row0, sem_row1, sem_sc0, sem_sc1):
  src_v = (src_v0, src_v1)
  dst_v = (dst_v0, dst_v1)
  w_v = (w_v0, w_v1)
  rows_v = (rows_v0, rows_v1)
  sc_v = (sc_v0, sc_v1)
  sem_sw = (sem_sw0, sem_sw1)
  sem_dst = (sem_dst0, sem_dst1)
  sem_row = (sem_row0, sem_row1)
  sem_sc = (sem_sc0, sem_sc1)

  cid = lax.axis_index("c")
  sid = lax.axis_index("s")
  wid = sid * _NC + cid
  base = wid * n_chunks

  def sw_copies(c, p):
    off = pl.ds((base + c) * _CHUNK, _CHUNK)
    return (
        pltpu.make_async_copy(src_hbm.at[off], src_v[p], sem_sw[p]),
        pltpu.make_async_copy(w_hbm.at[off], w_v[p], sem_sw[p]),
    )

  def dst_copy(c, p):
    off = pl.ds((base + c) * _CHUNK, _CHUNK)
    return pltpu.make_async_copy(dst_hbm.at[off], dst_v[p], sem_dst[p])

  def gather_copy(p):
    return pltpu.make_async_copy(x_hbm.at[src_v[p]], rows_v[p], sem_row[p])

  def scatter_copy(p):
    return pltpu.make_async_copy(sc_v[p], acc_sh.at[dst_v[p]], sem_sc[p])

  zeros = jnp.zeros((_LANES,), jnp.float32)

  def zfill(r, carry):
    for j in range(d // _LANES):
      zrow_v[r, j * _LANES:(j + 1) * _LANES] = zeros
    return carry

  lax.fori_loop(0, _ZBLK, zfill, 0)

  def zero_copy(k):
    sl = pl.ds(sid * rows_per_tile + k * _ZBLK, _ZBLK)
    return pltpu.make_async_copy(zrow_v, acc_sh.at[sl], sem_sw0)

  def zstart(k, carry):
    zero_copy(k).start()
    return carry

  def zdrain(k, carry):
    zero_copy(k).wait()
    return carry

  lax.fori_loop(0, rows_per_tile // _ZBLK, zstart, 0)
  lax.fori_loop(0, rows_per_tile // _ZBLK, zdrain, 0)
  plsc.subcore_barrier()

  def scale(p):
    def group_body(eg, c2):
      wvec = w_v[p][pl.ds(eg * _LANES, _LANES)]
      for l in range(_LANES):
        e = eg * _LANES + l
        w = wvec[l]
        for j in range(d // _LANES):
          sl = pl.ds(j * _LANES, _LANES)
          sc_v[p][e, sl] = rows_v[p][e, sl] * w
      return c2

    lax.fori_loop(0, _CHUNK // _LANES, group_body, 0)

  def _maybe(static, cond, fn):
    if static:
      if cond:
        fn()
    else:
      pl.when(cond)(fn)

  def iteration(g, p, first, has_next, has_next2):
    static = isinstance(g, int)
    q = 1 - p
    if not first:
      _maybe(static, g >= 2, lambda: scatter_copy(p).wait())
    gather_copy(p).wait()
    dst_copy(g, p).start()

    def do_next():
      for cp in sw_copies(g + 1, q):
        cp.wait()
      gather_copy(q).start()

    if has_next:
      _maybe(static, g + 1 < n_chunks, do_next)
    scale(p)
    dst_copy(g, p).wait()
    scatter_copy(p).start(add=True)

    def do_next2():
      for cp in sw_copies(g + 2, p):
        cp.start()

    if has_next2:
      _maybe(static, g + 2 < n_chunks, do_next2)

  for cp in sw_copies(0, 0):
    cp.start()
  for cp in sw_copies(1, 1):
    cp.start()
  for cp in sw_copies(0, 0):
    cp.wait()
  gather_copy(0).start()

  def pair_body(t, carry):
    g = 2 * t
    iteration(g, 0, False, True, True)
    iteration(g + 1, 1, False, True, True)
    return carry

  lax.fori_loop(0, n_chunks // 2, pair_body, 0)
  if n_chunks % 2:
    iteration(n_chunks - 1, 0, False, False, False)
  scatter_copy(1 - (n_chunks - 1) % 2).wait()
  scatter_copy((n_chunks - 1) % 2).wait()
  plsc.subcore_barrier()

  osl = pl.ds(sid * rows_per_tile, rows_per_tile)
  pltpu.sync_copy(acc_sh.at[osl], out_hbm.at[cid].at[osl])


def _pad_to(n, q):
  return ((n + q - 1) // q) * q


def _sc_aggregate(x, src, dst, ew):
  n_nodes, d = x.shape
  e_pad = src.shape[0]
  n_chunks = e_pad // (_NW * _CHUNK)
  n_pad = _pad_to(n_nodes, _ZBLK * _NS)
  rows_per_tile = n_pad // _NS

  mesh = plsc.VectorSubcoreMesh(core_axis_name="c", subcore_axis_name="s")
  kern = pl.kernel(
      functools.partial(_spmm_body, n_pad, d, n_chunks, rows_per_tile),
      mesh=mesh,
      out_type=jax.ShapeDtypeStruct((_NC, n_pad, d), jnp.float32),
      scratch_types=[
          pltpu.VMEM((_CHUNK,), jnp.int32),
          pltpu.VMEM((_CHUNK,), jnp.int32),
          pltpu.VMEM((_CHUNK,), jnp.int32),
          pltpu.VMEM((_CHUNK,), jnp.int32),
          pltpu.VMEM((_CHUNK,), jnp.float32),
          pltpu.VMEM((_CHUNK,), jnp.float32),
          pltpu.VMEM((_CHUNK, d), jnp.float32),
          pltpu.VMEM((_CHUNK, d), jnp.float32),
          pltpu.VMEM((_CHUNK, d), jnp.float32),
          pltpu.VMEM((_CHUNK, d), jnp.float32),
          pltpu.VMEM((_ZBLK, d), jnp.float32),
          pltpu.VMEM_SHARED((n_pad, d), jnp.float32),
          pltpu.SemaphoreType.DMA,
          pltpu.SemaphoreType.DMA,
          pltpu.SemaphoreType.DMA,
          pltpu.SemaphoreType.DMA,
          pltpu.SemaphoreType.DMA,
          pltpu.SemaphoreType.DMA,
          pltpu.SemaphoreType.DMA,
          pltpu.SemaphoreType.DMA,
      ],
  )
  return kern(x, src, dst, ew)


def _combine_matmul_kernel(p_ref, w_ref, b_ref, o_ref):
  s = p_ref[0] + p_ref[1]
  o_ref[...] = (
      jnp.dot(s, w_ref[...], preferred_element_type=jnp.float32) + b_ref[...]
  )


def _combine_matmul(partials, W, b, n_nodes):
  d_in = partials.shape[2]
  d_out = W.shape[1]
  blk = 1000
  grid = n_nodes // blk
  return pl.pallas_call(
      _combine_matmul_kernel,
      grid=(grid,),
      in_specs=[
          pl.BlockSpec((2, blk, d_in), lambda i: (0, i, 0)),
          pl.BlockSpec((d_in, d_out), lambda i: (0, 0)),
          pl.BlockSpec((1, d_out), lambda i: (0, 0)),
      ],
      out_specs=pl.BlockSpec((blk, d_out), lambda i: (i, 0)),
      out_shape=jax.ShapeDtypeStruct((n_nodes, d_out), jnp.float32),
  )(partials, W, b.reshape(1, d_out))


def kernel(x, edge_index, edge_weight, W, b):
  n_edges = edge_weight.shape[0]
  dst = edge_index[0].astype(jnp.int32)
  src = edge_index[1].astype(jnp.int32)
  ew = edge_weight.astype(jnp.float32)

  quantum = _NW * _CHUNK
  e_pad = _pad_to(n_edges, quantum)
  pad = e_pad - n_edges
  if pad:
    src = jnp.pad(src, (0, pad))
    dst = jnp.pad(dst, (0, pad))
    ew = jnp.pad(ew, (0, pad))

  partials = _sc_aggregate(x, src, dst, ew)
  return _combine_matmul(partials, W, b, x.shape[0])
